# baseline (speedup 1.0000x reference)
import jax
import jax.numpy as jnp
from jax.experimental import pallas as pl

B, N, D = 8, 2048, 256
H = N // 2


def _sgconv_block(x_ref, adj_a_ref, adj_b_ref, w_ref, b_ref, out_ref):
    a = adj_a_ref[0]
    bcols = adj_b_ref[0]
    deg = jnp.sum(jnp.abs(a), axis=1) + jnp.sum(jnp.abs(bcols), axis=1)
    norm = jax.lax.rsqrt(deg + 1e-6)[:, None]
    support = jnp.dot(x_ref[0], w_ref[...], preferred_element_type=jnp.float32)
    tmp = support * norm
    out = jnp.dot(a, tmp[:H], preferred_element_type=jnp.float32)
    out += jnp.dot(bcols, tmp[H:], preferred_element_type=jnp.float32)
    out_ref[0] = jnp.maximum(out * norm + b_ref[...], 0.0)


def kernel(x, adj, W, b):
    b2d = b.reshape(1, D)
    return pl.pallas_call(
        _sgconv_block,
        grid=(B,),
        in_specs=[
            pl.BlockSpec((1, N, D), lambda i: (i, 0, 0)),
            pl.BlockSpec((1, N, H), lambda i: (i, 0, 0)),
            pl.BlockSpec((1, N, H), lambda i: (i, 0, 1)),
            pl.BlockSpec((D, D), lambda i: (0, 0)),
            pl.BlockSpec((1, D), lambda i: (0, 0)),
        ],
        out_specs=pl.BlockSpec((1, N, D), lambda i: (i, 0, 0)),
        out_shape=jax.ShapeDtypeStruct((B, N, D), jnp.float32),
    )(x, adj, adj, W, b2d)


# speedup vs baseline: 1.0189x; 1.0189x over previous
import jax
import jax.numpy as jnp
from jax.experimental import pallas as pl

B, N, D = 8, 2048, 256
H = N // 2


def _sgconv_block(x_ref, adj_a_ref, adj_b_ref, w_ref, b_ref, out_ref):
    a = adj_a_ref[0]
    bb = adj_b_ref[0]
    deg_a = jnp.sum(jnp.abs(a), axis=1)
    deg_b = jnp.sum(jnp.abs(bb), axis=1)
    norm = jax.lax.rsqrt(jnp.concatenate([deg_a, deg_b]) + 1e-6)[:, None]
    support = jnp.dot(x_ref[0], w_ref[...], preferred_element_type=jnp.float32)
    tmp = support * norm
    out_a = jnp.dot(a, tmp, preferred_element_type=jnp.float32)
    out_b = jnp.dot(bb, tmp, preferred_element_type=jnp.float32)
    out = jnp.concatenate([out_a, out_b], axis=0) * norm
    out_ref[0] = jnp.maximum(out + b_ref[...], 0.0)


def kernel(x, adj, W, b):
    b2d = b.reshape(1, D)
    return pl.pallas_call(
        _sgconv_block,
        grid=(B,),
        in_specs=[
            pl.BlockSpec((1, N, D), lambda i: (i, 0, 0)),
            pl.BlockSpec((1, H, N), lambda i: (i, 0, 0)),
            pl.BlockSpec((1, H, N), lambda i: (i, 1, 0)),
            pl.BlockSpec((D, D), lambda i: (0, 0)),
            pl.BlockSpec((1, D), lambda i: (0, 0)),
        ],
        out_specs=pl.BlockSpec((1, N, D), lambda i: (i, 0, 0)),
        out_shape=jax.ShapeDtypeStruct((B, N, D), jnp.float32),
    )(x, adj, adj, W, b2d)
